# Initial kernel scaffold; baseline (speedup 1.0000x reference)
import jax
import jax.numpy as jnp
from jax.experimental import pallas as pl


def kernel(input_ids, combined_embedding_weight):
    raise NotImplementedError("write your pallas kernel here")


# speedup vs baseline: 1.3142x; 1.3142x over previous
import functools

import jax
import jax.numpy as jnp
from jax import lax
from jax.experimental import pallas as pl
from jax.experimental.pallas import tpu as pltpu
from jax.experimental.pallas import tpu_sc as plsc

DIM = 1024
BATCH = 16384
NUM_CORES = 2
NUM_SUBCORES = 16
NUM_WORKERS = NUM_CORES * NUM_SUBCORES
ROWS_PER_WORKER = BATCH // NUM_WORKERS
CHUNK = 32
NUM_CHUNKS = ROWS_PER_WORKER // CHUNK


def _gather_body(idx_hbm, table_hbm, out_hbm, idx_v, rows0, rows1,
                 gsem0, gsem1, ssem0, ssem1):
    wid = lax.axis_index("s") * NUM_CORES + lax.axis_index("c")
    base = wid * ROWS_PER_WORKER
    pltpu.sync_copy(idx_hbm.at[wid], idx_v)

    bufs = ((rows0, gsem0, ssem0), (rows1, gsem1, ssem1))
    pending_store = [None, None]
    for c in range(NUM_CHUNKS):
        rows, gsem, ssem = bufs[c % 2]
        if pending_store[c % 2] is not None:
            pending_store[c % 2].wait()
        gather = pltpu.make_async_copy(table_hbm.at[idx_v.at[c]], rows, gsem)
        gather.start()
        gather.wait()
        store = pltpu.make_async_copy(
            rows, out_hbm.at[pl.ds(base + c * CHUNK, CHUNK)], ssem)
        store.start()
        pending_store[c % 2] = store
    for p in pending_store:
        if p is not None:
            p.wait()


@functools.partial(jax.jit, static_argnames=())
def _gather(idx, table):
    mesh = plsc.VectorSubcoreMesh(core_axis_name="c", subcore_axis_name="s")
    return pl.kernel(
        _gather_body,
        out_type=jax.ShapeDtypeStruct((BATCH, DIM), jnp.float32),
        mesh=mesh,
        scratch_types=[
            pltpu.VMEM((NUM_CHUNKS, CHUNK), jnp.int32),
            pltpu.VMEM((CHUNK, DIM), jnp.float32),
            pltpu.VMEM((CHUNK, DIM), jnp.float32),
            pltpu.SemaphoreType.DMA,
            pltpu.SemaphoreType.DMA,
            pltpu.SemaphoreType.DMA,
            pltpu.SemaphoreType.DMA,
        ],
    )(idx, table)


def kernel(input_ids, combined_embedding_weight):
    idx = input_ids.astype(jnp.int32).reshape(NUM_WORKERS, NUM_CHUNKS, CHUNK)
    out = _gather(idx, combined_embedding_weight)
    return out[..., None, None]


# speedup vs baseline: 1.3597x; 1.0346x over previous
import functools

import jax
import jax.numpy as jnp
from jax import lax
from jax.experimental import pallas as pl
from jax.experimental.pallas import tpu as pltpu
from jax.experimental.pallas import tpu_sc as plsc

DIM = 1024
BATCH = 16384
NUM_CORES = 2
NUM_SUBCORES = 16
NUM_WORKERS = NUM_CORES * NUM_SUBCORES
ROWS_PER_WORKER = BATCH // NUM_WORKERS
CHUNK = 32
NUM_CHUNKS = ROWS_PER_WORKER // CHUNK
NBUF = 3


def _gather_body(idx_hbm, table_hbm, out_hbm, idx_v, *rest):
    rows = rest[:NBUF]
    gsems = rest[NBUF:2 * NBUF]
    ssems = rest[2 * NBUF:3 * NBUF]
    wid = lax.axis_index("s") * NUM_CORES + lax.axis_index("c")
    base = wid * ROWS_PER_WORKER
    pltpu.sync_copy(idx_hbm.at[wid], idx_v)

    pending_gather = [None] * NBUF
    pending_store = [None] * NBUF

    def start_gather(c):
        b = c % NBUF
        if pending_store[b] is not None:
            pending_store[b].wait()
            pending_store[b] = None
        g = pltpu.make_async_copy(table_hbm.at[idx_v.at[c]], rows[b], gsems[b])
        g.start()
        pending_gather[b] = g

    def finish_chunk(c):
        b = c % NBUF
        pending_gather[b].wait()
        pending_gather[b] = None
        s = pltpu.make_async_copy(
            rows[b], out_hbm.at[pl.ds(base + c * CHUNK, CHUNK)], ssems[b])
        s.start()
        pending_store[b] = s

    for c in range(NBUF - 1):
        start_gather(c)
    for c in range(NBUF - 1, NUM_CHUNKS):
        start_gather(c)
        finish_chunk(c - (NBUF - 1))
    for c in range(NUM_CHUNKS - (NBUF - 1), NUM_CHUNKS):
        finish_chunk(c)
    for p in pending_store:
        if p is not None:
            p.wait()


@functools.partial(jax.jit, static_argnames=())
def _gather(idx, table):
    mesh = plsc.VectorSubcoreMesh(core_axis_name="c", subcore_axis_name="s")
    return pl.kernel(
        _gather_body,
        out_type=jax.ShapeDtypeStruct((BATCH, DIM), jnp.float32),
        mesh=mesh,
        scratch_types=(
            [pltpu.VMEM((NUM_CHUNKS, CHUNK), jnp.int32)]
            + [pltpu.VMEM((CHUNK, DIM), jnp.float32)] * NBUF
            + [pltpu.SemaphoreType.DMA] * (2 * NBUF)
        ),
    )(idx, table)


def kernel(input_ids, combined_embedding_weight):
    idx = input_ids.astype(jnp.int32).reshape(NUM_WORKERS, NUM_CHUNKS, CHUNK)
    out = _gather(idx, combined_embedding_weight)
    return out[..., None, None]
